# baseline (speedup 1.0000x reference)
import functools

import jax
import jax.numpy as jnp
from jax import lax
from jax.experimental import pallas as pl
from jax.experimental.pallas import tpu as pltpu
from jax.experimental.pallas import tpu_sc as plsc

B = 16384
D = 32
L = 16
NC = 2
NS = 16
NW = NC * NS
BPW = B // NW
ICH = 128
NCH = BPW // ICH

_mesh = plsc.VectorSubcoreMesh(core_axis_name="c", subcore_axis_name="s")


@functools.partial(
    pl.kernel,
    mesh=_mesh,
    out_type=jax.ShapeDtypeStruct((B,), jnp.float32),
    compiler_params=pltpu.CompilerParams(
        needs_layout_passes=False, use_tc_tiling_on_sc=False),
    scratch_types=[
        pltpu.VMEM((NCH, ICH), jnp.int32),
        pltpu.VMEM((NCH, ICH), jnp.int32),
        pltpu.VMEM((BPW, D), jnp.float32),
        pltpu.VMEM((BPW, D), jnp.float32),
        pltpu.VMEM((BPW,), jnp.float32),
        pltpu.SemaphoreType.DMA,
    ],
)
def _cfd_sc(user_hbm, item_hbm, uemb_hbm, iemb_hbm, out_hbm,
            uidx_v, iidx_v, urows_v, irows_v, out_v, sem):
    wid = lax.axis_index("s") * NC + lax.axis_index("c")
    base = wid * BPW

    pltpu.sync_copy(user_hbm.at[wid], uidx_v)
    pltpu.sync_copy(item_hbm.at[wid], iidx_v)

    copies = []
    for j in range(NCH):
        copies.append(pltpu.async_copy(
            uemb_hbm.at[uidx_v.at[j]], urows_v.at[pl.ds(j * ICH, ICH)], sem))
        copies.append(pltpu.async_copy(
            iemb_hbm.at[iidx_v.at[j]], irows_v.at[pl.ds(j * ICH, ICH)], sem))
    for cp in copies:
        cp.wait()

    lane = lax.iota(jnp.int32, L)

    def chunk_body(c, carry):
        bidx = c * L + lane
        acc = jnp.zeros((L,), jnp.float32)
        for f in range(D):
            fidx = jnp.full((L,), f, jnp.int32)
            uv = plsc.load_gather(urows_v, [bidx, fidx])
            iv = plsc.load_gather(irows_v, [bidx, fidx])
            acc = acc + uv * iv
        out_v[pl.ds(c * L, L)] = 1.0 / (1.0 + jnp.exp(-acc))
        return carry

    lax.fori_loop(0, BPW // L, chunk_body, 0)

    pltpu.sync_copy(out_v, out_hbm.at[pl.ds(base, BPW)])


def kernel(user, item, user_emb, item_emb):
    user_r = user.reshape(NW, NCH, ICH)
    item_r = item.reshape(NW, NCH, ICH)
    return _cfd_sc(user_r, item_r, user_emb, item_emb)


# speedup vs baseline: 1.4870x; 1.4870x over previous
import functools

import jax
import jax.numpy as jnp
from jax import lax
from jax.experimental import pallas as pl
from jax.experimental.pallas import tpu as pltpu
from jax.experimental.pallas import tpu_sc as plsc

B = 16384
D = 32
L = 16
NC = 2
NS = 16
NW = NC * NS
BPW = B // NW
H = 256

_mesh = plsc.VectorSubcoreMesh(core_axis_name="c", subcore_axis_name="s")


@functools.partial(
    pl.kernel,
    mesh=_mesh,
    out_type=jax.ShapeDtypeStruct((B,), jnp.float32),
    compiler_params=pltpu.CompilerParams(
        needs_layout_passes=False, use_tc_tiling_on_sc=True),
    scratch_types=[
        pltpu.VMEM((BPW,), jnp.int32),
        pltpu.VMEM((BPW,), jnp.int32),
        pltpu.VMEM((H, D), jnp.float32),
        pltpu.VMEM((H, D), jnp.float32),
        pltpu.VMEM((BPW,), jnp.float32),
        pltpu.SemaphoreType.DMA,
    ],
)
def _cfd_sc(user_hbm, item_hbm, uemb_hbm, iemb_hbm, out_hbm,
            uidx_v, iidx_v, urows_v, irows_v, out_v, sem):
    wid = lax.axis_index("s") * NC + lax.axis_index("c")
    base = wid * BPW

    pltpu.sync_copy(user_hbm.at[pl.ds(base, BPW)], uidx_v)
    pltpu.sync_copy(item_hbm.at[pl.ds(base, BPW)], iidx_v)

    lane = lax.iota(jnp.int32, L)

    for h in range(BPW // H):
        def issue_group(g, carry):
            gbase = h * H + g * L
            uvec = uidx_v[pl.ds(gbase, L)]
            ivec = iidx_v[pl.ds(gbase, L)]
            for j in range(L):
                pltpu.async_copy(
                    uemb_hbm.at[pl.ds(uvec[j], 1)],
                    urows_v.at[pl.ds(g * L + j, 1)], sem)
                pltpu.async_copy(
                    iemb_hbm.at[pl.ds(ivec[j], 1)],
                    irows_v.at[pl.ds(g * L + j, 1)], sem)
            return carry

        lax.fori_loop(0, H // L, issue_group, 0)

        def drain_group(g, carry):
            for j in range(L):
                pltpu.make_async_copy(
                    uemb_hbm.at[pl.ds(0, 1)],
                    urows_v.at[pl.ds(g * L + j, 1)], sem).wait()
                pltpu.make_async_copy(
                    iemb_hbm.at[pl.ds(0, 1)],
                    irows_v.at[pl.ds(g * L + j, 1)], sem).wait()
            return carry

        lax.fori_loop(0, H // L, drain_group, 0)

        def chunk_body(c, carry):
            bidx = c * L + lane
            acc = jnp.zeros((L,), jnp.float32)
            for f in range(D):
                fidx = jnp.full((L,), f, jnp.int32)
                uv = plsc.load_gather(urows_v, [bidx, fidx])
                iv = plsc.load_gather(irows_v, [bidx, fidx])
                acc = acc + uv * iv
            out_v[pl.ds(h * H + c * L, L)] = 1.0 / (1.0 + jnp.exp(-acc))
            return carry

        lax.fori_loop(0, H // L, chunk_body, 0)

    pltpu.sync_copy(out_v, out_hbm.at[pl.ds(base, BPW)])


def kernel(user, item, user_emb, item_emb):
    return _cfd_sc(user, item, user_emb, item_emb)
